# baseline (speedup 1.0000x reference)
import functools

import jax
import jax.numpy as jnp
from jax import lax
from jax.experimental import pallas as pl
from jax.experimental.pallas import tpu as pltpu, tpu_sc as plsc

HIDDEN = 128
M_NODE = 128
RBLK = 64
NC, NS = 2, 16
NW = NC * NS
STARTS = (0, 32, 48, 56, 120)
PCHUNK = 128


def _sc_emb_sum(table, cate_sc, n_pairs):
    n_chunks = cate_sc.shape[1]
    per_w = n_chunks * PCHUNK
    mesh = plsc.VectorSubcoreMesh(core_axis_name="c", subcore_axis_name="s")

    @functools.partial(
        pl.kernel, mesh=mesh,
        out_type=jax.ShapeDtypeStruct((n_pairs, HIDDEN), jnp.float32),
        scratch_types=[
            pltpu.VMEM((5, PCHUNK), jnp.int32),
            pltpu.VMEM((5, PCHUNK), jnp.int32),
            pltpu.VMEM((PCHUNK,), jnp.int32),
            pltpu.VMEM((PCHUNK, HIDDEN), jnp.float32),
            pltpu.VMEM((PCHUNK, HIDDEN), jnp.float32),
            pltpu.VMEM_SHARED((NS * PCHUNK, HIDDEN), jnp.float32),
            pltpu.SemaphoreType.DMA,
            pltpu.SemaphoreType.DMA,
        ],
    )
    def k(table_hbm, cate_hbm, out_hbm, cate_v, idx_v, sidx_v, gbuf0, gbuf1,
          acc, gsem, ssem):
        gbufs = (gbuf0, gbuf1)
        cid = lax.axis_index("c")
        sid = lax.axis_index("s")
        wid = sid * NC + cid
        for c8 in range(PCHUNK // 16):
            sidx_v[pl.ds(c8 * 16, 16)] = (
                lax.iota(jnp.int32, 16) + (c8 * 16) + sid * PCHUNK)

        def chunk_body(ch, _):
            pltpu.sync_copy(cate_hbm.at[wid, ch], cate_v)
            for f in range(5):
                for c8 in range(PCHUNK // 16):
                    sl = pl.ds(c8 * 16, 16)
                    idx_v[f, sl] = cate_v[f, sl] + (STARTS[f] + 248 * wid)
            g = pltpu.async_copy(table_hbm.at[idx_v.at[0]], gbufs[0], gsem)
            for f in range(5):
                g.wait()
                if f < 4:
                    g = pltpu.async_copy(table_hbm.at[idx_v.at[f + 1]],
                                         gbufs[(f + 1) % 2], gsem)
                s = pltpu.async_copy(gbufs[f % 2], acc.at[sidx_v], ssem,
                                     add=(f > 0))
                s.wait()
            base = wid * per_w + ch * PCHUNK
            pltpu.sync_copy(acc.at[pl.ds(sid * PCHUNK, PCHUNK)],
                            out_hbm.at[pl.ds(base, PCHUNK)])
            return ()

        lax.fori_loop(0, n_chunks, chunk_body, ())

    return k(table, cate_sc)


def _tc_body(emb_ref, flt_ref, w_ref, pvec_ref, out_ref):
    i = pl.program_id(1)
    rblk = out_ref.shape[1]
    m = rblk * M_NODE
    f32 = jnp.float32

    wc = w_ref[...] - jnp.mean(w_ref[...], axis=1, keepdims=True)
    bc = pvec_ref[2] - jnp.mean(pvec_ref[2])

    ones_hh = jnp.ones((HIDDEN, HIDDEN), f32)

    emb = emb_ref[0].reshape(m, HIDDEN)
    mean_rep = jnp.dot(emb, ones_hh, preferred_element_type=f32) * (1.0 / HIDDEN)
    xc = emb - mean_rep
    vc = jnp.dot(xc * xc, ones_hh, preferred_element_type=f32)
    rc = lax.rsqrt(vc * (1.0 / HIDDEN) + 1e-5)
    cate_emb = xc * (rc * pvec_ref[0]) + pvec_ref[1]

    xf = flt_ref[0].reshape(m, 8).astype(f32)
    fc = jnp.dot(xf, wc, preferred_element_type=f32) + bc
    vf = jnp.dot(fc * fc, ones_hh, preferred_element_type=f32)
    rf = lax.rsqrt(vf * (1.0 / HIDDEN) + 1e-5)
    flt_emb = fc * (rf * pvec_ref[3]) + pvec_ref[4]

    val = (cate_emb + flt_emb).reshape(rblk, M_NODE, HIDDEN)

    ridx = lax.broadcasted_iota(jnp.int32, (rblk, M_NODE, 1), 0) + i * rblk
    cidx = lax.broadcasted_iota(jnp.int32, (rblk, M_NODE, 1), 1)
    border = (ridx == 0) | (cidx == 0)
    out_ref[0] = jnp.where(border, pvec_ref[5][None, None, :], val)


def kernel(structure_feat_cate, structure_feat_float, emb_table, ln_cate_g,
           ln_cate_b, W_float, b_float, ln_float_g, ln_float_b,
           virtual_edge_emb):
    B = structure_feat_cate.shape[0]
    n_pairs = B * M_NODE * M_NODE
    per_w = n_pairs // NW
    n_chunks = per_w // PCHUNK

    cate_pad = jnp.pad(structure_feat_cate, ((0, 0), (1, 0), (1, 0), (0, 0)))
    cf = cate_pad.transpose(3, 0, 1, 2).reshape(5, n_pairs)
    cate_sc = cf.reshape(5, NW, n_chunks, PCHUNK).transpose(1, 2, 0, 3)
    flt_pad = jnp.pad(structure_feat_float, ((0, 0), (1, 0), (1, 0), (0, 0)))

    pvec = jnp.stack([ln_cate_g, ln_cate_b, b_float, ln_float_g, ln_float_b,
                      virtual_edge_emb.reshape(HIDDEN)], axis=0)

    table_rep = jnp.broadcast_to(emb_table, (NW,) + emb_table.shape).reshape(
        NW * emb_table.shape[0], HIDDEN)
    emb_sum = _sc_emb_sum(table_rep, cate_sc, n_pairs)
    emb4 = emb_sum.reshape(B, M_NODE, M_NODE, HIDDEN)

    grid = (B, M_NODE // RBLK)
    out = pl.pallas_call(
        _tc_body,
        grid=grid,
        in_specs=[
            pl.BlockSpec((1, RBLK, M_NODE, HIDDEN), lambda b, i: (b, i, 0, 0)),
            pl.BlockSpec((1, RBLK, M_NODE, 8), lambda b, i: (b, i, 0, 0)),
            pl.BlockSpec((8, HIDDEN), lambda b, i: (0, 0)),
            pl.BlockSpec((6, HIDDEN), lambda b, i: (0, 0)),
        ],
        out_specs=pl.BlockSpec((1, RBLK, M_NODE, HIDDEN),
                               lambda b, i: (b, i, 0, 0)),
        out_shape=jax.ShapeDtypeStruct((B, M_NODE, M_NODE, HIDDEN),
                                       jnp.float32),
    )(emb4, flt_pad, W_float, pvec)
    return out


# speedup vs baseline: 4.5298x; 4.5298x over previous
import functools

import jax
import jax.numpy as jnp
from jax import lax
from jax.experimental import pallas as pl

HIDDEN = 128
M_NODE = 128
RBLK = 64


def _body(cate_ref, flt_ref, tp_ref, w_ref, pvec_ref, out_ref):
    i = pl.program_id(1)
    rblk = out_ref.shape[1]
    m = rblk * M_NODE
    f32 = jnp.float32

    tc = tp_ref[...] - jnp.mean(tp_ref[...], axis=1, keepdims=True)
    wc = w_ref[...] - jnp.mean(w_ref[...], axis=1, keepdims=True)
    bc = pvec_ref[2] - jnp.mean(pvec_ref[2])

    iota8 = lax.broadcasted_iota(jnp.int32, (8, m), 0)
    bands = []
    for f in range(5):
        idx_f = cate_ref[0, f, 0][None, :]
        bands.append((idx_f == iota8).astype(f32))
    oht = jnp.concatenate(bands, axis=0)

    ones_hh = jnp.ones((HIDDEN, HIDDEN), f32)

    xc = lax.dot_general(oht, tc, (((0,), (0,)), ((), ())),
                         preferred_element_type=f32)
    vc = jnp.dot(xc * xc, ones_hh, preferred_element_type=f32)
    rc = lax.rsqrt(vc * (1.0 / HIDDEN) + 1e-5)
    cate_emb = xc * (rc * pvec_ref[0]) + pvec_ref[1]

    xf = flt_ref[0].reshape(m, 8).astype(f32)
    fc = jnp.dot(xf, wc, preferred_element_type=f32) + bc
    vf = jnp.dot(fc * fc, ones_hh, preferred_element_type=f32)
    rf = lax.rsqrt(vf * (1.0 / HIDDEN) + 1e-5)
    flt_emb = fc * (rf * pvec_ref[3]) + pvec_ref[4]

    val = (cate_emb + flt_emb).reshape(rblk, M_NODE, HIDDEN)

    ridx = lax.broadcasted_iota(jnp.int32, (rblk, M_NODE, 1), 0) + i * rblk
    cidx = lax.broadcasted_iota(jnp.int32, (rblk, M_NODE, 1), 1)
    border = (ridx == 0) | (cidx == 0)
    out_ref[0] = jnp.where(border, pvec_ref[5][None, None, :], val)


def kernel(structure_feat_cate, structure_feat_float, emb_table, ln_cate_g,
           ln_cate_b, W_float, b_float, ln_float_g, ln_float_b,
           virtual_edge_emb):
    B = structure_feat_cate.shape[0]

    starts = (0, 32, 48, 56, 120)
    tp = jnp.concatenate([emb_table[s:s + 8] for s in starts], axis=0)

    cate_pad = jnp.pad(structure_feat_cate, ((0, 0), (1, 0), (1, 0), (0, 0)))
    cate_t = cate_pad.transpose(0, 3, 1, 2).reshape(
        B, 5, 1, M_NODE * M_NODE)
    flt_pad = jnp.pad(structure_feat_float, ((0, 0), (1, 0), (1, 0), (0, 0)))

    pvec = jnp.stack([ln_cate_g, ln_cate_b, b_float, ln_float_g, ln_float_b,
                      virtual_edge_emb.reshape(HIDDEN)], axis=0)

    grid = (B, M_NODE // RBLK)
    out = pl.pallas_call(
        _body,
        grid=grid,
        in_specs=[
            pl.BlockSpec((1, 5, 1, RBLK * M_NODE), lambda b, i: (b, 0, 0, i)),
            pl.BlockSpec((1, RBLK, M_NODE, 8), lambda b, i: (b, i, 0, 0)),
            pl.BlockSpec((40, HIDDEN), lambda b, i: (0, 0)),
            pl.BlockSpec((8, HIDDEN), lambda b, i: (0, 0)),
            pl.BlockSpec((6, HIDDEN), lambda b, i: (0, 0)),
        ],
        out_specs=pl.BlockSpec((1, RBLK, M_NODE, HIDDEN),
                               lambda b, i: (b, i, 0, 0)),
        out_shape=jax.ShapeDtypeStruct((B, M_NODE, M_NODE, HIDDEN),
                                       jnp.float32),
    )(cate_t, flt_pad, tp, W_float, pvec)
    return out
